# Initial kernel scaffold; baseline (speedup 1.0000x reference)
import jax
import jax.numpy as jnp
from jax.experimental import pallas as pl


def kernel(node_invariant, batch, W1, b1, W2, b2):
    raise NotImplementedError("write your pallas kernel here")


# speedup vs baseline: 1.2983x; 1.2983x over previous
import functools

import jax
import jax.numpy as jnp
from jax import lax
from jax.experimental import pallas as pl
from jax.experimental.pallas import tpu as pltpu
from jax.experimental.pallas import tpu_sc as plsc

_N_NODES = 100000
_NODE_DIM = 128
_HIDDEN_DIM = 64
_NUM_SEGMENTS = 512


_ROWS = 2000
_NUM_BLOCKS = _N_NODES // _ROWS


def _mlp_body(x_ref, w1_ref, b1_ref, w2_ref, b2_ref, out_ref):
    x = x_ref[...]
    h = jnp.dot(x, w1_ref[...], preferred_element_type=jnp.float32)
    h = h + b1_ref[...]
    h = h * jax.nn.sigmoid(h)
    e = jnp.dot(h, w2_ref[...], preferred_element_type=jnp.float32)
    out_ref[...] = e + b2_ref[0, 0]


def _mlp(x, W1, b1, W2, b2):
    return pl.pallas_call(
        _mlp_body,
        grid=(_NUM_BLOCKS,),
        in_specs=[
            pl.BlockSpec((_ROWS, _NODE_DIM), lambda i: (i, 0)),
            pl.BlockSpec((_NODE_DIM, _HIDDEN_DIM), lambda i: (0, 0)),
            pl.BlockSpec((1, _HIDDEN_DIM), lambda i: (0, 0)),
            pl.BlockSpec((_HIDDEN_DIM, 1), lambda i: (0, 0)),
            pl.BlockSpec((1, 1), lambda i: (0, 0)),
        ],
        out_specs=pl.BlockSpec((_ROWS, 1), lambda i: (i, 0)),
        out_shape=jax.ShapeDtypeStruct((_N_NODES, 1), jnp.float32),
    )(x, W1, b1.reshape(1, _HIDDEN_DIM), W2, b2.reshape(1, 1))


_NW = 16
_CHUNK = 6256
_N_PAD = _NW * _CHUNK
_VECS = _CHUNK // 16
_SEG_PER_W = _NUM_SEGMENTS // _NW
_LANES = 16


def _segsum_body(e_hbm, b_hbm, out_hbm, e_v, b_v, accf_v, acc_v, tmp_v, res_v, shared):
    wid = lax.axis_index("s")
    base = wid * _CHUNK
    pltpu.sync_copy(e_hbm.at[pl.ds(base, _CHUNK)], e_v)
    pltpu.sync_copy(b_hbm.at[pl.ds(base, _CHUNK)], b_v)
    zero = jnp.zeros((16,), jnp.float32)
    lane_off = lax.iota(jnp.int32, 16) * _NUM_SEGMENTS

    def zbody(j, carry):
        accf_v[pl.ds(j * 16, 16)] = zero
        return carry

    lax.fori_loop(0, _LANES * _NUM_SEGMENTS // 16, zbody, 0)

    def body(i, carry):
        idx = b_v[pl.ds(i * 16, 16)] + lane_off
        v = e_v[pl.ds(i * 16, 16)]
        plsc.addupdate_scatter(accf_v, [idx], v)
        return carry

    lax.fori_loop(0, _VECS, body, 0)

    def rbody(j, carry):
        s = zero
        for r in range(_LANES):
            s = s + accf_v[pl.ds(r * _NUM_SEGMENTS + j * 16, 16)]
        acc_v[pl.ds(j * 16, 16)] = s
        return carry

    lax.fori_loop(0, _NUM_SEGMENTS // 16, rbody, 0)

    pltpu.sync_copy(acc_v, shared.at[wid])
    plsc.subcore_barrier()

    col = wid * _SEG_PER_W
    for t in range(_NW):
        pltpu.sync_copy(shared.at[t, pl.ds(col, _SEG_PER_W)], tmp_v.at[t])
    for q in range(_SEG_PER_W // 16):
        s = zero
        for t in range(_NW):
            s = s + tmp_v[t, pl.ds(q * 16, 16)]
        res_v[pl.ds(q * 16, 16)] = s
    pltpu.sync_copy(res_v, out_hbm.at[pl.ds(col, _SEG_PER_W)])


def _segment_sum(e_pad, b_pad):
    mesh = plsc.VectorSubcoreMesh(
        core_axis_name="c", subcore_axis_name="s", num_cores=1
    )
    return pl.kernel(
        _segsum_body,
        mesh=mesh,
        out_type=jax.ShapeDtypeStruct((_NUM_SEGMENTS,), jnp.float32),
        scratch_types=[
            pltpu.VMEM((_CHUNK,), jnp.float32),
            pltpu.VMEM((_CHUNK,), jnp.int32),
            pltpu.VMEM((_LANES * _NUM_SEGMENTS,), jnp.float32),
            pltpu.VMEM((_NUM_SEGMENTS,), jnp.float32),
            pltpu.VMEM((_NW, _SEG_PER_W), jnp.float32),
            pltpu.VMEM((_SEG_PER_W,), jnp.float32),
            pltpu.VMEM_SHARED((_NW, _NUM_SEGMENTS), jnp.float32),
        ],
        compiler_params=pltpu.CompilerParams(needs_layout_passes=False),
    )(e_pad, b_pad)


def kernel(node_invariant, batch, W1, b1, W2, b2):
    atomic_energies = _mlp(node_invariant, W1, b1, W2, b2)
    e_pad = jnp.pad(atomic_energies.reshape(_N_NODES), (0, _N_PAD - _N_NODES))
    b_pad = jnp.pad(batch.astype(jnp.int32), (0, _N_PAD - _N_NODES))
    total = _segment_sum(e_pad, b_pad)
    return (total.reshape(_NUM_SEGMENTS, 1), atomic_energies)
